# baseline (speedup 1.0000x reference)
import functools

import jax
import jax.numpy as jnp
from jax import lax
from jax.experimental import pallas as pl
from jax.experimental.pallas import tpu as pltpu
from jax.experimental.pallas import tpu_sc as plsc

B, C, H, W = 16, 16, 56, 56
RF = 2
OH = (H - RF) + 1
OW = (W - RF) + 1
P = OH * OW
R, K, S = 2, 32, 4
CHW = C * H * W
ROW = P * S
ROW_PAD = 12160
NCHUNK = ROW_PAD // 16
NBLK = R * K
HALF = NBLK // 2


def _sc_gather_kernel(x_hbm, bases_hbm, off_hbm, out_hbm,
                      xb, offv, basesv, ob0, ob1, sem0, sem1):
    b = lax.axis_index("s")
    h = lax.axis_index("c")

    pltpu.sync_copy(x_hbm.at[b], xb)
    pltpu.sync_copy(off_hbm, offv)
    pltpu.sync_copy(bases_hbm, basesv)
    zv = basesv[pl.ds(0, 16)] - basesv[pl.ds(0, 16)]
    sv = basesv[pl.ds(NBLK * 16, 16)]

    obufs = (ob0, ob1)
    sems = (sem0, sem1)

    def do_block(i, obuf):
        rk = h * HALF + i
        base_vec = basesv[pl.ds(rk * 16, 16)]

        @plsc.parallel_loop(0, NCHUNK, 1, unroll=8)
        def chunk(j):
            packed = offv[pl.ds(j * 16, 16)]
            row = jax.lax.shift_right_logical(packed, 16)
            idx = (packed & 0xFFFF) + base_vec
            vals = plsc.load_gather(xb, [zv, zv, idx])
            plsc.store_scatter(obuf, [row, sv], vals)

    def dst(blk):
        rk = h * HALF + blk
        return out_hbm.at[b, rk // K, rk - (rk // K) * K]

    def loop_body(i, _):
        for par in range(2):
            obuf, sem = obufs[par], sems[par]
            blk = i * 2 + par
            @pl.when(i > 0)
            def _wait():
                pltpu.make_async_copy(
                    obuf.at[pl.ds(0, P)], dst(blk - 2), sem).wait()
            do_block(blk, obuf)
            pltpu.make_async_copy(
                obuf.at[pl.ds(0, P)], dst(blk), sem).start()
        return _

    lax.fori_loop(0, HALF // 2, loop_body, None)

    for par in range(2):
        pltpu.make_async_copy(
            obufs[par].at[pl.ds(0, P)], dst(HALF - 2 + par), sems[par]).wait()


def kernel(x, flat_idx):
    bases = flat_idx[:, :, 0, :].reshape(NBLK, 1, S).astype(jnp.int32)
    bvecs = jnp.tile(bases, (1, 16 // S, 1)).reshape(NBLK * 16)
    bvecs = jnp.concatenate(
        [bvecs, jnp.tile(jnp.arange(S, dtype=jnp.int32), 16 // S)])
    q = jnp.arange(ROW_PAD, dtype=jnp.int32)
    p = q // S
    off = jnp.where(p < P, (p // OW) * W + (p - (p // OW) * OW), 0)
    off = (off | (p << 16)).astype(jnp.int32)

    mesh = plsc.VectorSubcoreMesh(core_axis_name="c", subcore_axis_name="s")
    f = functools.partial(
        pl.kernel,
        out_type=jax.ShapeDtypeStruct((B, R, K, P, S), jnp.float32),
        mesh=mesh,
        scratch_types=[
            pltpu.VMEM((C, H, W), jnp.float32),
            pltpu.VMEM((ROW_PAD,), jnp.int32),
            pltpu.VMEM((NBLK * 16 + 16,), jnp.int32),
            pltpu.VMEM((ROW_PAD // S, S), jnp.float32),
            pltpu.VMEM((ROW_PAD // S, S), jnp.float32),
            pltpu.SemaphoreType.DMA,
            pltpu.SemaphoreType.DMA,
        ],
        compiler_params=pltpu.CompilerParams(
            needs_layout_passes=False, use_tc_tiling_on_sc=False),
    )(_sc_gather_kernel)
    return f(x, bvecs, off)


# speedup vs baseline: 13.7733x; 13.7733x over previous
import functools

import jax
import jax.numpy as jnp
from jax import lax
from jax.experimental import pallas as pl
from jax.experimental.pallas import tpu as pltpu
from jax.experimental.pallas import tpu_sc as plsc

B, C, H, W = 16, 16, 56, 56
RF = 2
OH = (H - RF) + 1
OW = (W - RF) + 1
P = OH * OW
R, K, S = 2, 32, 4
CHW = C * H * W
PPAD = 3072
NCHUNK = PPAD // 16
NBLK = R * K
HALF = NBLK // 2


def _sc_gather_kernel(x_hbm, bases_hbm, off_hbm, out_hbm,
                      xb, offv, basesv, ob0, ob1, sem0, sem1):
    b = lax.axis_index("s")
    h = lax.axis_index("c")

    pltpu.sync_copy(x_hbm.at[b], xb)
    pltpu.sync_copy(off_hbm, offv)
    pltpu.sync_copy(bases_hbm, basesv)
    zv = basesv[pl.ds(0, 16)] - basesv[pl.ds(0, 16)]

    obufs = (ob0, ob1)
    sems = (sem0, sem1)

    def do_block(i, obuf):
        rk = h * HALF + i
        for s in range(S):
            base_vec = basesv[pl.ds((rk * S + s) * 16, 16)]

            @plsc.parallel_loop(0, NCHUNK, 1, unroll=8)
            def chunk(j):
                idx = offv[pl.ds(j * 16, 16)] + base_vec
                obuf[pl.ds(s * PPAD + j * 16, 16)] = plsc.load_gather(
                    xb, [zv, zv, idx])

    def block_dmas(obuf, sem, blk):
        rk = h * HALF + blk
        r = rk // K
        k = rk - r * K
        return [pltpu.make_async_copy(
                    obuf.at[pl.ds(s * PPAD, P)], out_hbm.at[b, r, k, s], sem)
                for s in range(S)]

    def loop_body(i, _):
        for par in range(2):
            obuf, sem = obufs[par], sems[par]
            blk = i * 2 + par
            @pl.when(i > 0)
            def _wait():
                for c in block_dmas(obuf, sem, blk - 2):
                    c.wait()
            do_block(blk, obuf)
            for c in block_dmas(obuf, sem, blk):
                c.start()
        return _

    lax.fori_loop(0, HALF // 2, loop_body, None)

    for par in range(2):
        for c in block_dmas(obufs[par], sems[par], HALF - 2 + par):
            c.wait()


def kernel(x, flat_idx):
    bases = flat_idx[:, :, 0, :].reshape(NBLK * S, 1).astype(jnp.int32)
    bvecs = jnp.tile(bases, (1, 16)).reshape(NBLK * S * 16)
    p = jnp.arange(PPAD, dtype=jnp.int32)
    off = jnp.where(p < P, (p // OW) * W + (p - (p // OW) * OW), 0)
    off = off.astype(jnp.int32)

    mesh = plsc.VectorSubcoreMesh(core_axis_name="c", subcore_axis_name="s")
    f = functools.partial(
        pl.kernel,
        out_type=jax.ShapeDtypeStruct((B, R, K, S, P), jnp.float32),
        mesh=mesh,
        scratch_types=[
            pltpu.VMEM((C, H, W), jnp.float32),
            pltpu.VMEM((PPAD,), jnp.int32),
            pltpu.VMEM((NBLK * S * 16,), jnp.int32),
            pltpu.VMEM((S * PPAD,), jnp.float32),
            pltpu.VMEM((S * PPAD,), jnp.float32),
            pltpu.SemaphoreType.DMA,
            pltpu.SemaphoreType.DMA,
        ],
        compiler_params=pltpu.CompilerParams(
            needs_layout_passes=False, use_tc_tiling_on_sc=False),
    )(_sc_gather_kernel)
    out_sp = f(x, bvecs, off)
    return jnp.transpose(out_sp, (0, 1, 2, 4, 3))
